# Initial kernel scaffold; baseline (speedup 1.0000x reference)
import jax
import jax.numpy as jnp
from jax.experimental import pallas as pl


def kernel(users_all, items_all, user_emb, item_emb, edge_index, norm):
    raise NotImplementedError("write your pallas kernel here")


# speedup vs baseline: 2.6864x; 2.6864x over previous
import functools

import jax
import jax.numpy as jnp
from jax import lax
from jax.experimental import pallas as pl
from jax.experimental.pallas import tpu as pltpu
from jax.experimental.pallas import tpu_sc as plsc

_N_USERS = 50000
_N_ITEMS = 50000
_N = _N_USERS + _N_ITEMS
_D = 64
_H = 32
_E = 1200000
_EH = _E // 2
_CHUNK = 96
_NCHUNKS = _EH // _CHUNK
_NTILES = 16
_WB = 80
_NBLK = _N_USERS // _WB


def _layer_body(last, *refs):
  if last:
    (x_lo, x_hi, s_lo, s_hi, src, dst, norm, sn_lo, sn_hi,
     acc, src_s, dst_s, norm_s, rows, accv, sv, sem) = refs
    xn_lo = xn_hi = None
  else:
    (x_lo, x_hi, s_lo, s_hi, src, dst, norm, xn_lo, xn_hi, sn_lo, sn_hi,
     acc, src_s, dst_s, norm_s, rows, accv, sv, sem) = refs
  cid = lax.axis_index("c")
  tid = lax.axis_index("s")
  e_base = jnp.where(cid == 0, _EH, 0)
  node_base = cid * _N_USERS
  dst_sub = jnp.full((16,), cid * _N_USERS, jnp.int32)
  wmul = jnp.full((16,), (1.0 / 3.0) if last else 1.0, jnp.float32)
  zeros16 = jnp.zeros((16,), jnp.float32)

  n_chunks = jnp.where(tid < (_NCHUNKS % _NTILES),
                       _NCHUNKS // _NTILES + 1, _NCHUNKS // _NTILES)
  n_blocks = jnp.where(tid < (_NBLK % _NTILES),
                       _NBLK // _NTILES + 1, _NBLK // _NTILES)

  for x_in, s_in, x_out, s_out in ((x_lo, s_lo, xn_lo, sn_lo),
                                   (x_hi, s_hi, xn_hi, sn_hi)):
    for i in range(_CHUNK):
      rows[i, pl.ds(0, 16)] = zeros16
      rows[i, pl.ds(16, 16)] = zeros16

    def _zero_step(j, _):
      blk = tid + j * _NTILES
      pltpu.sync_copy(rows.at[pl.ds(0, _WB)], acc.at[pl.ds(blk * _WB, _WB)])
      return _
    lax.fori_loop(0, n_blocks, _zero_step, None)
    plsc.subcore_barrier()

    def _edge_step(i, _):
      c = tid + i * _NTILES
      b = e_base + c * _CHUNK
      pltpu.sync_copy(src.at[pl.ds(b, _CHUNK)], src_s.at[0])
      pltpu.sync_copy(dst.at[pl.ds(b, _CHUNK)], dst_s.at[0])
      pltpu.sync_copy(norm.at[pl.ds(b, _CHUNK)], norm_s.at[0])
      for k in range(_CHUNK // 16):
        v = dst_s[0, pl.ds(k * 16, 16)]
        dst_s[0, pl.ds(k * 16, 16)] = v - dst_sub
      pltpu.async_copy(x_in.at[src_s.at[0]], rows, sem).wait()
      for g in range(_CHUNK // 16):
        nv = norm_s[0, pl.ds(g * 16, 16)]
        for j in range(16):
          i2 = g * 16 + j
          nb = jnp.full((16,), nv[j], jnp.float32)
          rows[i2, pl.ds(0, 16)] = rows[i2, pl.ds(0, 16)] * nb
          rows[i2, pl.ds(16, 16)] = rows[i2, pl.ds(16, 16)] * nb
      pltpu.sync_copy(rows, acc.at[dst_s.at[0]], add=True)
      return _
    lax.fori_loop(0, n_chunks, _edge_step, None)
    plsc.subcore_barrier()

    def _wb_step(j, _):
      r0 = (tid + j * _NTILES) * _WB
      g0 = node_base + r0
      pltpu.sync_copy(acc.at[pl.ds(r0, _WB)], accv)
      pltpu.sync_copy(s_in.at[pl.ds(g0, _WB)], sv)

      def _row(r, _2):
        a0 = accv[r, pl.ds(0, 16)]
        a1 = accv[r, pl.ds(16, 16)]
        sv[r, pl.ds(0, 16)] = (sv[r, pl.ds(0, 16)] + a0) * wmul
        sv[r, pl.ds(16, 16)] = (sv[r, pl.ds(16, 16)] + a1) * wmul
        return _2
      lax.fori_loop(0, _WB, _row, None)
      if not last:
        pltpu.sync_copy(accv, x_out.at[pl.ds(g0, _WB)])
      pltpu.sync_copy(sv, s_out.at[pl.ds(g0, _WB)])
      return _
    lax.fori_loop(0, n_blocks, _wb_step, None)
    plsc.subcore_barrier()


def _make_layer(last):
  half = jax.ShapeDtypeStruct((_N, _H), jnp.float32)
  n_out = 2 if last else 4
  mesh = plsc.VectorSubcoreMesh(core_axis_name="c", subcore_axis_name="s",
                                num_cores=2, num_subcores=_NTILES)
  return pl.kernel(
      functools.partial(_layer_body, last),
      out_type=tuple(half for _ in range(n_out)),
      mesh=mesh,
      compiler_params=pltpu.CompilerParams(use_tc_tiling_on_sc=False),
      scratch_types=[
          pltpu.VMEM_SHARED((_N_USERS, _H), jnp.float32),
          pltpu.VMEM((1, _CHUNK), jnp.int32),
          pltpu.VMEM((1, _CHUNK), jnp.int32),
          pltpu.VMEM((1, _CHUNK), jnp.float32),
          pltpu.VMEM((_CHUNK, _H), jnp.float32),
          pltpu.VMEM((_WB, _H), jnp.float32),
          pltpu.VMEM((_WB, _H), jnp.float32),
          pltpu.SemaphoreType.DMA,
      ],
  )


def kernel(users_all, items_all, user_emb, item_emb, edge_index, norm):
  u0 = jnp.take(user_emb, users_all, axis=0)
  i0 = jnp.take(item_emb, items_all, axis=0)
  x0 = jnp.concatenate([u0, i0], axis=0)
  x0_lo = x0[:, :_H]
  x0_hi = x0[:, _H:]
  src = edge_index[0]
  dst = edge_index[1]

  layer1 = _make_layer(False)
  layer2 = _make_layer(True)
  x1_lo, x1_hi, s1_lo, s1_hi = layer1(
      x0_lo, x0_hi, x0_lo, x0_hi, src, dst, norm)
  s2_lo, s2_hi = layer2(x1_lo, x1_hi, s1_lo, s1_hi, src, dst, norm)

  h_user = jnp.concatenate([s2_lo[:_N_USERS], s2_hi[:_N_USERS]], axis=1)
  h_item = jnp.concatenate([s2_lo[_N_USERS:], s2_hi[_N_USERS:]], axis=1)
  return (h_user, h_item)


# speedup vs baseline: 6.0399x; 2.2483x over previous
import functools

import jax
import jax.numpy as jnp
from jax import lax
from jax.experimental import pallas as pl
from jax.experimental.pallas import tpu as pltpu
from jax.experimental.pallas import tpu_sc as plsc

_N_USERS = 50000
_N_ITEMS = 50000
_N = _N_USERS + _N_ITEMS
_D = 64
_H = 32
_E = 1200000
_EH = _E // 2
_CHUNK = 128
_NTILES = 16
_CPT = 294
_NCHUNKS = _CPT * _NTILES
_EPAD = _NCHUNKS * _CHUNK
_WB = 80
_NBLK = _N_USERS // _WB


def _layer_body(last, *refs):
  if last:
    (x_lo, x_hi, s_lo, s_hi, packed, sn_lo, sn_hi,
     acc, idx_s, rows, zbuf, accv, sv, sem_g0, sem_g1) = refs
    xn_lo = xn_hi = None
  else:
    (x_lo, x_hi, s_lo, s_hi, packed, xn_lo, xn_hi, sn_lo, sn_hi,
     acc, idx_s, rows, zbuf, accv, sv, sem_g0, sem_g1) = refs
  sems = (sem_g0, sem_g1)
  cid = lax.axis_index("c")
  tid = lax.axis_index("s")
  node_base = cid * _N_USERS
  dst_sub = jnp.full((16,), cid * _N_USERS, jnp.int32)
  wmul = jnp.full((16,), (1.0 / 3.0) if last else 1.0, jnp.float32)
  zeros16 = jnp.zeros((16,), jnp.float32)

  n_blocks = jnp.where(tid < (_NBLK % _NTILES),
                       _NBLK // _NTILES + 1, _NBLK // _NTILES)

  for i in range(_WB):
    zbuf[i, pl.ds(0, 16)] = zeros16
    zbuf[i, pl.ds(16, 16)] = zeros16

  for x_in, s_in, x_out, s_out in ((x_lo, s_lo, xn_lo, sn_lo),
                                   (x_hi, s_hi, xn_hi, sn_hi)):
    def _zero_step(j, _):
      blk = tid + j * _NTILES
      pltpu.sync_copy(zbuf, acc.at[pl.ds(blk * _WB, _WB)])
      return _
    lax.fori_loop(0, n_blocks, _zero_step, None)
    plsc.subcore_barrier()

    def _stage(i_local, slot):
      c = tid + i_local * _NTILES
      pltpu.sync_copy(packed.at[cid, c], idx_s.at[slot])
      for k in range(_CHUNK // 16):
        v = idx_s[slot, 1, pl.ds(k * 16, 16)]
        idx_s[slot, 1, pl.ds(k * 16, 16)] = v - dst_sub

    def _gather_start(slot):
      pltpu.async_copy(x_in.at[idx_s.at[slot, 0]], rows.at[slot], sems[slot])

    def _gather_wait(slot):
      pltpu.make_async_copy(x_in.at[idx_s.at[slot, 0]], rows.at[slot],
                            sems[slot]).wait()

    def _process(slot):
      _gather_wait(slot)

      def _scale(g, _):
        nv = plsc.bitcast(idx_s[slot, 2, pl.ds(g * 16, 16)], jnp.float32)
        for j in range(16):
          r = g * 16 + j
          nb = jnp.full((16,), nv[j], jnp.float32)
          rows[slot, r, pl.ds(0, 16)] = rows[slot, r, pl.ds(0, 16)] * nb
          rows[slot, r, pl.ds(16, 16)] = rows[slot, r, pl.ds(16, 16)] * nb
        return _
      lax.fori_loop(0, _CHUNK // 16, _scale, None)
      pltpu.sync_copy(rows.at[slot], acc.at[idx_s.at[slot, 1]], add=True)

    _stage(0, 0)
    _gather_start(0)

    def _edge_step(k, _):
      _stage(2 * k + 1, 1)
      _gather_start(1)
      _process(0)
      nxt = lax.rem(2 * k + 2, _CPT)
      _stage(nxt, 0)
      _gather_start(0)
      _process(1)
      return _
    lax.fori_loop(0, _CPT // 2, _edge_step, None)
    _gather_wait(0)
    plsc.subcore_barrier()

    def _wb_step(j, _):
      r0 = (tid + j * _NTILES) * _WB
      g0 = node_base + r0
      pltpu.sync_copy(acc.at[pl.ds(r0, _WB)], accv)
      pltpu.sync_copy(s_in.at[pl.ds(g0, _WB)], sv)

      def _row(r, _2):
        a0 = accv[r, pl.ds(0, 16)]
        a1 = accv[r, pl.ds(16, 16)]
        sv[r, pl.ds(0, 16)] = (sv[r, pl.ds(0, 16)] + a0) * wmul
        sv[r, pl.ds(16, 16)] = (sv[r, pl.ds(16, 16)] + a1) * wmul
        return _2
      lax.fori_loop(0, _WB, _row, None)
      if not last:
        pltpu.sync_copy(accv, x_out.at[pl.ds(g0, _WB)])
      pltpu.sync_copy(sv, s_out.at[pl.ds(g0, _WB)])
      return _
    lax.fori_loop(0, n_blocks, _wb_step, None)
    plsc.subcore_barrier()


def _make_layer(last):
  half = jax.ShapeDtypeStruct((_N, _H), jnp.float32)
  n_out = 2 if last else 4
  mesh = plsc.VectorSubcoreMesh(core_axis_name="c", subcore_axis_name="s",
                                num_cores=2, num_subcores=_NTILES)
  return pl.kernel(
      functools.partial(_layer_body, last),
      out_type=tuple(half for _ in range(n_out)),
      mesh=mesh,
      compiler_params=pltpu.CompilerParams(use_tc_tiling_on_sc=False,
                                           needs_layout_passes=False),
      scratch_types=[
          pltpu.VMEM_SHARED((_N_USERS, _H), jnp.float32),
          pltpu.VMEM((2, 3, _CHUNK), jnp.int32),
          pltpu.VMEM((2, _CHUNK, _H), jnp.float32),
          pltpu.VMEM((_WB, _H), jnp.float32),
          pltpu.VMEM((_WB, _H), jnp.float32),
          pltpu.VMEM((_WB, _H), jnp.float32),
          pltpu.SemaphoreType.DMA,
          pltpu.SemaphoreType.DMA,
      ],
  )


def _pack_half(src, dst, nrm, dst_base):
  npad = _EPAD - _EH
  pad_src = jnp.arange(npad, dtype=jnp.int32) % _N
  pad_dst = jnp.arange(npad, dtype=jnp.int32) % _N_USERS + dst_base
  s = jnp.concatenate([src, pad_src])
  d = jnp.concatenate([dst, pad_dst])
  n = jnp.concatenate([nrm, jnp.zeros((npad,), jnp.float32)])
  trip = jnp.stack([s, d, lax.bitcast_convert_type(n, jnp.int32)])
  return trip.reshape(3, _NCHUNKS, _CHUNK).transpose(1, 0, 2)


def kernel(users_all, items_all, user_emb, item_emb, edge_index, norm):
  u0 = jnp.take(user_emb, users_all, axis=0)
  i0 = jnp.take(item_emb, items_all, axis=0)
  x0 = jnp.concatenate([u0, i0], axis=0)
  x0_lo = x0[:, :_H]
  x0_hi = x0[:, _H:]
  src = edge_index[0]
  dst = edge_index[1]
  packed = jnp.stack([
      _pack_half(src[_EH:], dst[_EH:], norm[_EH:], 0),
      _pack_half(src[:_EH], dst[:_EH], norm[:_EH], _N_USERS),
  ])

  layer1 = _make_layer(False)
  layer2 = _make_layer(True)
  x1_lo, x1_hi, s1_lo, s1_hi = layer1(
      x0_lo, x0_hi, x0_lo, x0_hi, packed)
  s2_lo, s2_hi = layer2(x1_lo, x1_hi, s1_lo, s1_hi, packed)

  h_user = jnp.concatenate([s2_lo[:_N_USERS], s2_hi[:_N_USERS]], axis=1)
  h_item = jnp.concatenate([s2_lo[_N_USERS:], s2_hi[_N_USERS:]], axis=1)
  return (h_user, h_item)


# speedup vs baseline: 8.0063x; 1.3256x over previous
import functools

import jax
import jax.numpy as jnp
from jax import lax
from jax.experimental import pallas as pl
from jax.experimental.pallas import tpu as pltpu
from jax.experimental.pallas import tpu_sc as plsc

_N_USERS = 50000
_N_ITEMS = 50000
_N = _N_USERS + _N_ITEMS
_D = 64
_H = 32
_E = 1200000
_EH = _E // 2
_CHUNK = 128
_NTILES = 16
_CPT = 294
_NCHUNKS = _CPT * _NTILES
_EPAD = _NCHUNKS * _CHUNK
_WB = 80
_NBLK = _N_USERS // _WB
_IDXB = 3 * _CHUNK * 4
_GATB = _CHUNK * _H * 4
_SCB = _CHUNK * _H * 4


def _layer_body(last, *refs):
  if last:
    (x_lo, x_hi, s_lo, s_hi, packed, sn_lo, sn_hi,
     acc, idx_s, rows, zbuf, accv, sv, *sems) = refs
    xn_lo = xn_hi = None
  else:
    (x_lo, x_hi, s_lo, s_hi, packed, xn_lo, xn_hi, sn_lo, sn_hi,
     acc, idx_s, rows, zbuf, accv, sv, *sems) = refs
  sem_i, sem_g, sem_sc = sems[0:4], sems[4:8], sems[8:12]
  cid = lax.axis_index("c")
  tid = lax.axis_index("s")
  node_base = cid * _N_USERS
  dst_sub = jnp.full((16,), cid * _N_USERS, jnp.int32)
  wmul = jnp.full((16,), (1.0 / 3.0) if last else 1.0, jnp.float32)
  zeros16 = jnp.zeros((16,), jnp.float32)

  n_blocks = jnp.where(tid < (_NBLK % _NTILES),
                       _NBLK // _NTILES + 1, _NBLK // _NTILES)

  for i in range(_WB):
    zbuf[i, pl.ds(0, 16)] = zeros16
    zbuf[i, pl.ds(16, 16)] = zeros16

  for x_in, s_in, x_out, s_out in ((x_lo, s_lo, xn_lo, sn_lo),
                                   (x_hi, s_hi, xn_hi, sn_hi)):
    def _zero_step(j, _):
      blk = tid + j * _NTILES
      pltpu.sync_copy(zbuf, acc.at[pl.ds(blk * _WB, _WB)])
      return _
    lax.fori_loop(0, n_blocks, _zero_step, None)
    plsc.subcore_barrier()

    def _idx_start(i_local, slot):
      c = tid + lax.rem(i_local, _CPT) * _NTILES
      pltpu.async_copy(packed.at[cid, c], idx_s.at[slot], sem_i[slot])

    def _idx_wait(slot):
      pltpu.make_async_copy(packed.at[cid, tid], idx_s.at[slot],
                            sem_i[slot]).wait()

    def _gather_wait(slot):
      pltpu.make_async_copy(x_in.at[idx_s.at[slot, 0]], rows.at[slot],
                            sem_g[slot]).wait()

    def _scatter_wait(slot):
      pltpu.make_async_copy(rows.at[slot], acc.at[idx_s.at[slot, 1]],
                            sem_sc[slot]).wait()

    def _rebase(slot):
      for k in range(_CHUNK // 16):
        v = idx_s[slot, 1, pl.ds(k * 16, 16)]
        idx_s[slot, 1, pl.ds(k * 16, 16)] = v - dst_sub

    def _gather_start(slot):
      pltpu.async_copy(x_in.at[idx_s.at[slot, 0]], rows.at[slot],
                       sem_g[slot])

    def _scale(slot):
      def _sc(g, _):
        nv = plsc.bitcast(idx_s[slot, 2, pl.ds(g * 16, 16)], jnp.float32)
        for j in range(16):
          r = g * 16 + j
          nb = jnp.full((16,), nv[j], jnp.float32)
          rows[slot, r, pl.ds(0, 16)] = rows[slot, r, pl.ds(0, 16)] * nb
          rows[slot, r, pl.ds(16, 16)] = rows[slot, r, pl.ds(16, 16)] * nb
        return _
      lax.fori_loop(0, _CHUNK // 16, _sc, None)

    def _body(i_local, s, first=False):
      s1, s2 = (s + 1) % 4, (s + 2) % 4
      if not first:
        _scatter_wait(s2)
      _idx_start(i_local + 2, s2)
      _idx_wait(s1)
      _rebase(s1)
      _gather_start(s1)
      _gather_wait(s)
      _scale(s)
      pltpu.async_copy(rows.at[s], acc.at[idx_s.at[s, 1]], sem_sc[s],
                       add=True)

    _idx_start(0, 0)
    _idx_start(1, 1)
    _idx_wait(0)
    _rebase(0)
    _gather_start(0)
    _body(0, 0, first=True)
    _body(1, 1, first=True)

    def _edge_step(k, _):
      _body(4 * k + 2, 2)
      _body(4 * k + 3, 3)
      _body(4 * k + 4, 0)
      _body(4 * k + 5, 1)
      return _
    lax.fori_loop(0, (_CPT - 2) // 4, _edge_step, None)
    _idx_wait(3)
    _gather_wait(2)
    _scatter_wait(0)
    _scatter_wait(1)
    plsc.subcore_barrier()

    def _wb_step(j, _):
      r0 = (tid + j * _NTILES) * _WB
      g0 = node_base + r0
      pltpu.sync_copy(acc.at[pl.ds(r0, _WB)], accv)
      pltpu.sync_copy(s_in.at[pl.ds(g0, _WB)], sv)

      def _row(r, _2):
        a0 = accv[r, pl.ds(0, 16)]
        a1 = accv[r, pl.ds(16, 16)]
        sv[r, pl.ds(0, 16)] = (sv[r, pl.ds(0, 16)] + a0) * wmul
        sv[r, pl.ds(16, 16)] = (sv[r, pl.ds(16, 16)] + a1) * wmul
        return _2
      lax.fori_loop(0, _WB, _row, None)
      if not last:
        pltpu.sync_copy(accv, x_out.at[pl.ds(g0, _WB)])
      pltpu.sync_copy(sv, s_out.at[pl.ds(g0, _WB)])
      return _
    lax.fori_loop(0, n_blocks, _wb_step, None)
    plsc.subcore_barrier()


def _make_layer(last):
  half = jax.ShapeDtypeStruct((_N, _H), jnp.float32)
  n_out = 2 if last else 4
  mesh = plsc.VectorSubcoreMesh(core_axis_name="c", subcore_axis_name="s",
                                num_cores=2, num_subcores=_NTILES)
  return pl.kernel(
      functools.partial(_layer_body, last),
      out_type=tuple(half for _ in range(n_out)),
      mesh=mesh,
      compiler_params=pltpu.CompilerParams(use_tc_tiling_on_sc=False,
                                           needs_layout_passes=False),
      scratch_types=[
          pltpu.VMEM_SHARED((_N_USERS, _H), jnp.float32),
          pltpu.VMEM((4, 3, _CHUNK), jnp.int32),
          pltpu.VMEM((4, _CHUNK, _H), jnp.float32),
          pltpu.VMEM((_WB, _H), jnp.float32),
          pltpu.VMEM((_WB, _H), jnp.float32),
          pltpu.VMEM((_WB, _H), jnp.float32),
      ] + [pltpu.SemaphoreType.DMA] * 12,
  )


def _pack_half(src, dst, nrm, dst_base):
  npad = _EPAD - _EH
  pad_src = jnp.arange(npad, dtype=jnp.int32) % _N
  pad_dst = jnp.arange(npad, dtype=jnp.int32) % _N_USERS + dst_base
  s = jnp.concatenate([src, pad_src])
  d = jnp.concatenate([dst, pad_dst])
  n = jnp.concatenate([nrm, jnp.zeros((npad,), jnp.float32)])
  trip = jnp.stack([s, d, lax.bitcast_convert_type(n, jnp.int32)])
  return trip.reshape(3, _NCHUNKS, _CHUNK).transpose(1, 0, 2)


def kernel(users_all, items_all, user_emb, item_emb, edge_index, norm):
  u0 = jnp.take(user_emb, users_all, axis=0)
  i0 = jnp.take(item_emb, items_all, axis=0)
  x0 = jnp.concatenate([u0, i0], axis=0)
  x0_lo = x0[:, :_H]
  x0_hi = x0[:, _H:]
  src = edge_index[0]
  dst = edge_index[1]
  packed = jnp.stack([
      _pack_half(src[_EH:], dst[_EH:], norm[_EH:], 0),
      _pack_half(src[:_EH], dst[:_EH], norm[:_EH], _N_USERS),
  ])

  layer1 = _make_layer(False)
  layer2 = _make_layer(True)
  x1_lo, x1_hi, s1_lo, s1_hi = layer1(
      x0_lo, x0_hi, x0_lo, x0_hi, packed)
  s2_lo, s2_hi = layer2(x1_lo, x1_hi, s1_lo, s1_hi, packed)

  h_user = jnp.concatenate([s2_lo[:_N_USERS], s2_hi[:_N_USERS]], axis=1)
  h_item = jnp.concatenate([s2_lo[_N_USERS:], s2_hi[_N_USERS:]], axis=1)
  return (h_user, h_item)


# speedup vs baseline: 8.4502x; 1.0554x over previous
import functools

import jax
import jax.numpy as jnp
from jax import lax
from jax.experimental import pallas as pl
from jax.experimental.pallas import tpu as pltpu
from jax.experimental.pallas import tpu_sc as plsc

_N_USERS = 50000
_N_ITEMS = 50000
_N = _N_USERS + _N_ITEMS
_D = 64
_H = 32
_E = 1200000
_EH = _E // 2
_CHUNK = 128
_NTILES = 16
_CPT = 294
_NCHUNKS = _CPT * _NTILES
_EPAD = _NCHUNKS * _CHUNK
_WB = 200
_NBLK = _N_USERS // _WB
_IDXB = 3 * _CHUNK * 4
_GATB = _CHUNK * _H * 4
_SCB = _CHUNK * _H * 4


def _layer_body(last, *refs):
  if last:
    (x_lo, x_hi, s_lo, s_hi, packed, sn_lo, sn_hi,
     acc, idx_s, rows, accv, sv, *sems) = refs
    xn_lo = xn_hi = None
  else:
    (x_lo, x_hi, s_lo, s_hi, packed, xn_lo, xn_hi, sn_lo, sn_hi,
     acc, idx_s, rows, accv, sv, *sems) = refs
  sem_i, sem_g, sem_sc = sems[0:4], sems[4:8], sems[8:12]
  cid = lax.axis_index("c")
  tid = lax.axis_index("s")
  node_base = cid * _N_USERS
  dst_sub = jnp.full((16,), cid * _N_USERS, jnp.int32)
  wmul = jnp.full((16,), (1.0 / 3.0) if last else 1.0, jnp.float32)
  zeros16 = jnp.zeros((16,), jnp.float32)

  n_blocks = jnp.where(tid < (_NBLK % _NTILES),
                       _NBLK // _NTILES + 1, _NBLK // _NTILES)

  for x_in, s_in, x_out, s_out in ((x_lo, s_lo, xn_lo, sn_lo),
                                   (x_hi, s_hi, xn_hi, sn_hi)):
    def _zfill(r, _):
      accv[r, pl.ds(0, 16)] = zeros16
      accv[r, pl.ds(16, 16)] = zeros16
      return _
    lax.fori_loop(0, _WB, _zfill, None)

    def _zero_step(j, _):
      blk = tid + j * _NTILES
      pltpu.sync_copy(accv, acc.at[pl.ds(blk * _WB, _WB)])
      return _
    lax.fori_loop(0, n_blocks, _zero_step, None)
    plsc.subcore_barrier()

    def _idx_start(i_local, slot):
      c = tid + lax.rem(i_local, _CPT) * _NTILES
      pltpu.async_copy(packed.at[cid, c], idx_s.at[slot], sem_i[slot])

    def _idx_wait(slot):
      pltpu.make_async_copy(packed.at[cid, tid], idx_s.at[slot],
                            sem_i[slot]).wait()

    def _gather_wait(slot):
      pltpu.make_async_copy(x_in.at[idx_s.at[slot, 0]], rows.at[slot],
                            sem_g[slot]).wait()

    def _scatter_wait(slot):
      pltpu.make_async_copy(rows.at[slot], acc.at[idx_s.at[slot, 1]],
                            sem_sc[slot]).wait()

    def _rebase(slot):
      for k in range(_CHUNK // 16):
        v = idx_s[slot, 1, pl.ds(k * 16, 16)]
        idx_s[slot, 1, pl.ds(k * 16, 16)] = v - dst_sub

    def _gather_start(slot):
      pltpu.async_copy(x_in.at[idx_s.at[slot, 0]], rows.at[slot],
                       sem_g[slot])

    def _scale(slot):
      def _sc(g, _):
        nv = plsc.bitcast(idx_s[slot, 2, pl.ds(g * 16, 16)], jnp.float32)
        for j in range(16):
          r = g * 16 + j
          nb = jnp.full((16,), nv[j], jnp.float32)
          rows[slot, r, pl.ds(0, 16)] = rows[slot, r, pl.ds(0, 16)] * nb
          rows[slot, r, pl.ds(16, 16)] = rows[slot, r, pl.ds(16, 16)] * nb
        return _
      lax.fori_loop(0, _CHUNK // 16, _sc, None)

    def _body(i_local, s, first=False):
      s1, s2 = (s + 1) % 4, (s + 2) % 4
      if not first:
        _scatter_wait(s2)
      _idx_start(i_local + 2, s2)
      _idx_wait(s1)
      _rebase(s1)
      _gather_start(s1)
      _gather_wait(s)
      _scale(s)
      pltpu.async_copy(rows.at[s], acc.at[idx_s.at[s, 1]], sem_sc[s],
                       add=True)

    _idx_start(0, 0)
    _idx_start(1, 1)
    _idx_wait(0)
    _rebase(0)
    _gather_start(0)
    _body(0, 0, first=True)
    _body(1, 1, first=True)

    def _edge_step(k, _):
      _body(4 * k + 2, 2)
      _body(4 * k + 3, 3)
      _body(4 * k + 4, 0)
      _body(4 * k + 5, 1)
      return _
    lax.fori_loop(0, (_CPT - 2) // 4, _edge_step, None)
    _idx_wait(3)
    _gather_wait(2)
    _scatter_wait(0)
    _scatter_wait(1)
    plsc.subcore_barrier()

    def _wb_step(j, _):
      r0 = (tid + j * _NTILES) * _WB
      g0 = node_base + r0
      pltpu.sync_copy(acc.at[pl.ds(r0, _WB)], accv)
      pltpu.sync_copy(s_in.at[pl.ds(g0, _WB)], sv)

      def _row(r, _2):
        a0 = accv[r, pl.ds(0, 16)]
        a1 = accv[r, pl.ds(16, 16)]
        sv[r, pl.ds(0, 16)] = (sv[r, pl.ds(0, 16)] + a0) * wmul
        sv[r, pl.ds(16, 16)] = (sv[r, pl.ds(16, 16)] + a1) * wmul
        return _2
      lax.fori_loop(0, _WB, _row, None)
      if not last:
        pltpu.sync_copy(accv, x_out.at[pl.ds(g0, _WB)])
      pltpu.sync_copy(sv, s_out.at[pl.ds(g0, _WB)])
      return _
    lax.fori_loop(0, n_blocks, _wb_step, None)
    plsc.subcore_barrier()


def _make_layer(last):
  half = jax.ShapeDtypeStruct((_N, _H), jnp.float32)
  n_out = 2 if last else 4
  mesh = plsc.VectorSubcoreMesh(core_axis_name="c", subcore_axis_name="s",
                                num_cores=2, num_subcores=_NTILES)
  return pl.kernel(
      functools.partial(_layer_body, last),
      out_type=tuple(half for _ in range(n_out)),
      mesh=mesh,
      compiler_params=pltpu.CompilerParams(use_tc_tiling_on_sc=False,
                                           needs_layout_passes=False),
      scratch_types=[
          pltpu.VMEM_SHARED((_N_USERS, _H), jnp.float32),
          pltpu.VMEM((4, 3, _CHUNK), jnp.int32),
          pltpu.VMEM((4, _CHUNK, _H), jnp.float32),
          pltpu.VMEM((_WB, _H), jnp.float32),
          pltpu.VMEM((_WB, _H), jnp.float32),
      ] + [pltpu.SemaphoreType.DMA] * 12,
  )


def _pack_half(src, dst, nrm, dst_base):
  npad = _EPAD - _EH
  pad_src = jnp.arange(npad, dtype=jnp.int32) % _N
  pad_dst = jnp.arange(npad, dtype=jnp.int32) % _N_USERS + dst_base
  s = jnp.concatenate([src, pad_src])
  d = jnp.concatenate([dst, pad_dst])
  n = jnp.concatenate([nrm, jnp.zeros((npad,), jnp.float32)])
  trip = jnp.stack([s, d, lax.bitcast_convert_type(n, jnp.int32)])
  return trip.reshape(3, _NCHUNKS, _CHUNK).transpose(1, 0, 2)


def kernel(users_all, items_all, user_emb, item_emb, edge_index, norm):
  u0 = jnp.take(user_emb, users_all, axis=0)
  i0 = jnp.take(item_emb, items_all, axis=0)
  x0 = jnp.concatenate([u0, i0], axis=0)
  x0_lo = x0[:, :_H]
  x0_hi = x0[:, _H:]
  src = edge_index[0]
  dst = edge_index[1]
  packed = jnp.stack([
      _pack_half(src[_EH:], dst[_EH:], norm[_EH:], 0),
      _pack_half(src[:_EH], dst[:_EH], norm[:_EH], _N_USERS),
  ])

  layer1 = _make_layer(False)
  layer2 = _make_layer(True)
  x1_lo, x1_hi, s1_lo, s1_hi = layer1(
      x0_lo, x0_hi, x0_lo, x0_hi, packed)
  s2_lo, s2_hi = layer2(x1_lo, x1_hi, s1_lo, s1_hi, packed)

  h_user = jnp.concatenate([s2_lo[:_N_USERS], s2_hi[:_N_USERS]], axis=1)
  h_item = jnp.concatenate([s2_lo[_N_USERS:], s2_hi[_N_USERS:]], axis=1)
  return (h_user, h_item)


# speedup vs baseline: 8.5104x; 1.0071x over previous
import functools

import jax
import jax.numpy as jnp
from jax import lax
from jax.experimental import pallas as pl
from jax.experimental.pallas import tpu as pltpu
from jax.experimental.pallas import tpu_sc as plsc

_N_USERS = 50000
_N_ITEMS = 50000
_N = _N_USERS + _N_ITEMS
_D = 64
_H = 32
_E = 1200000
_EH = _E // 2
_CHUNK = 128
_NTILES = 16
_CPT = 294
_NCHUNKS = _CPT * _NTILES
_EPAD = _NCHUNKS * _CHUNK
_WB = 200
_NBLK = _N_USERS // _WB
_IDXB = 3 * _CHUNK * 4
_GATB = _CHUNK * _H * 4
_SCB = _CHUNK * _H * 4


def _layer_body(last, *refs):
  if last:
    (x_lo, x_hi, s_lo, s_hi, packed, sn_lo, sn_hi,
     acc, idx_s, rows, accv, sv, *sems) = refs
    xn_lo = xn_hi = None
  else:
    (x_lo, x_hi, s_lo, s_hi, packed, xn_lo, xn_hi, sn_lo, sn_hi,
     acc, idx_s, rows, accv, sv, *sems) = refs
  sem_i, sem_g, sem_sc = sems[0:4], sems[4:8], sems[8:12]
  cid = lax.axis_index("c")
  tid = lax.axis_index("s")
  node_base = cid * _N_USERS
  dst_sub = jnp.full((16,), cid * _N_USERS, jnp.int32)
  wmul = jnp.full((16,), (1.0 / 3.0) if last else 1.0, jnp.float32)
  zeros16 = jnp.zeros((16,), jnp.float32)

  n_blocks = jnp.where(tid < (_NBLK % _NTILES),
                       _NBLK // _NTILES + 1, _NBLK // _NTILES)

  for x_in, s_in, x_out, s_out in ((x_lo, s_lo, xn_lo, sn_lo),
                                   (x_hi, s_hi, xn_hi, sn_hi)):
    def _zfill(r, _):
      accv[r, pl.ds(0, 16)] = zeros16
      accv[r, pl.ds(16, 16)] = zeros16
      return _
    lax.fori_loop(0, _WB, _zfill, None)

    def _zero_step(j, _):
      blk = tid + j * _NTILES
      pltpu.sync_copy(accv, acc.at[pl.ds(blk * _WB, _WB)])
      return _
    lax.fori_loop(0, n_blocks, _zero_step, None)
    plsc.subcore_barrier()

    def _idx_start(i_local, slot):
      c = tid + lax.rem(i_local, _CPT) * _NTILES
      pltpu.async_copy(packed.at[cid, c], idx_s.at[slot], sem_i[slot])

    def _idx_wait(slot):
      pltpu.make_async_copy(packed.at[cid, tid], idx_s.at[slot],
                            sem_i[slot]).wait()

    def _gather_wait(slot):
      pltpu.make_async_copy(x_in.at[idx_s.at[slot, 0]], rows.at[slot],
                            sem_g[slot]).wait()

    def _scatter_wait(slot):
      pltpu.make_async_copy(rows.at[slot], acc.at[idx_s.at[slot, 1]],
                            sem_sc[slot]).wait()

    def _rebase(slot):
      for k in range(_CHUNK // 16):
        v = idx_s[slot, 1, pl.ds(k * 16, 16)]
        idx_s[slot, 1, pl.ds(k * 16, 16)] = v - dst_sub

    def _gather_start(slot):
      pltpu.async_copy(x_in.at[idx_s.at[slot, 0]], rows.at[slot],
                       sem_g[slot])

    def _scale(slot):
      @plsc.parallel_loop(0, _CHUNK // 16, unroll=2)
      def _sc(g):
        nv = plsc.bitcast(idx_s[slot, 2, pl.ds(g * 16, 16)], jnp.float32)
        for j in range(16):
          r = g * 16 + j
          nb = jnp.full((16,), nv[j], jnp.float32)
          rows[slot, r, pl.ds(0, 16)] = rows[slot, r, pl.ds(0, 16)] * nb
          rows[slot, r, pl.ds(16, 16)] = rows[slot, r, pl.ds(16, 16)] * nb

    def _body(i_local, s, first=False):
      s1, s2 = (s + 1) % 4, (s + 2) % 4
      if not first:
        _scatter_wait(s2)
      _idx_start(i_local + 2, s2)
      _idx_wait(s1)
      _rebase(s1)
      _gather_start(s1)
      _gather_wait(s)
      _scale(s)
      pltpu.async_copy(rows.at[s], acc.at[idx_s.at[s, 1]], sem_sc[s],
                       add=True)

    _idx_start(0, 0)
    _idx_start(1, 1)
    _idx_wait(0)
    _rebase(0)
    _gather_start(0)
    _body(0, 0, first=True)
    _body(1, 1, first=True)

    def _edge_step(k, _):
      _body(4 * k + 2, 2)
      _body(4 * k + 3, 3)
      _body(4 * k + 4, 0)
      _body(4 * k + 5, 1)
      return _
    lax.fori_loop(0, (_CPT - 2) // 4, _edge_step, None)
    _idx_wait(3)
    _gather_wait(2)
    _scatter_wait(0)
    _scatter_wait(1)
    plsc.subcore_barrier()

    def _wb_step(j, _):
      r0 = (tid + j * _NTILES) * _WB
      g0 = node_base + r0
      pltpu.sync_copy(acc.at[pl.ds(r0, _WB)], accv)
      pltpu.sync_copy(s_in.at[pl.ds(g0, _WB)], sv)

      def _row(r, _2):
        a0 = accv[r, pl.ds(0, 16)]
        a1 = accv[r, pl.ds(16, 16)]
        sv[r, pl.ds(0, 16)] = (sv[r, pl.ds(0, 16)] + a0) * wmul
        sv[r, pl.ds(16, 16)] = (sv[r, pl.ds(16, 16)] + a1) * wmul
        return _2
      lax.fori_loop(0, _WB, _row, None)
      if not last:
        pltpu.sync_copy(accv, x_out.at[pl.ds(g0, _WB)])
      pltpu.sync_copy(sv, s_out.at[pl.ds(g0, _WB)])
      return _
    lax.fori_loop(0, n_blocks, _wb_step, None)
    plsc.subcore_barrier()


def _make_layer(last):
  half = jax.ShapeDtypeStruct((_N, _H), jnp.float32)
  n_out = 2 if last else 4
  mesh = plsc.VectorSubcoreMesh(core_axis_name="c", subcore_axis_name="s",
                                num_cores=2, num_subcores=_NTILES)
  return pl.kernel(
      functools.partial(_layer_body, last),
      out_type=tuple(half for _ in range(n_out)),
      mesh=mesh,
      compiler_params=pltpu.CompilerParams(use_tc_tiling_on_sc=False,
                                           needs_layout_passes=False),
      scratch_types=[
          pltpu.VMEM_SHARED((_N_USERS, _H), jnp.float32),
          pltpu.VMEM((4, 3, _CHUNK), jnp.int32),
          pltpu.VMEM((4, _CHUNK, _H), jnp.float32),
          pltpu.VMEM((_WB, _H), jnp.float32),
          pltpu.VMEM((_WB, _H), jnp.float32),
      ] + [pltpu.SemaphoreType.DMA] * 12,
  )


def _pack_half(src, dst, nrm, dst_base):
  npad = _EPAD - _EH
  pad_src = jnp.arange(npad, dtype=jnp.int32) % _N
  pad_dst = jnp.arange(npad, dtype=jnp.int32) % _N_USERS + dst_base
  s = jnp.concatenate([src, pad_src])
  d = jnp.concatenate([dst, pad_dst])
  n = jnp.concatenate([nrm, jnp.zeros((npad,), jnp.float32)])
  trip = jnp.stack([s, d, lax.bitcast_convert_type(n, jnp.int32)])
  return trip.reshape(3, _NCHUNKS, _CHUNK).transpose(1, 0, 2)


def kernel(users_all, items_all, user_emb, item_emb, edge_index, norm):
  u0 = jnp.take(user_emb, users_all, axis=0)
  i0 = jnp.take(item_emb, items_all, axis=0)
  x0 = jnp.concatenate([u0, i0], axis=0)
  x0_lo = x0[:, :_H]
  x0_hi = x0[:, _H:]
  src = edge_index[0]
  dst = edge_index[1]
  packed = jnp.stack([
      _pack_half(src[_EH:], dst[_EH:], norm[_EH:], 0),
      _pack_half(src[:_EH], dst[:_EH], norm[:_EH], _N_USERS),
  ])

  layer1 = _make_layer(False)
  layer2 = _make_layer(True)
  x1_lo, x1_hi, s1_lo, s1_hi = layer1(
      x0_lo, x0_hi, x0_lo, x0_hi, packed)
  s2_lo, s2_hi = layer2(x1_lo, x1_hi, s1_lo, s1_hi, packed)

  h_user = jnp.concatenate([s2_lo[:_N_USERS], s2_hi[:_N_USERS]], axis=1)
  h_item = jnp.concatenate([s2_lo[_N_USERS:], s2_hi[_N_USERS:]], axis=1)
  return (h_user, h_item)
